# baseline (speedup 1.0000x reference)
import functools

import jax
import jax.numpy as jnp
from jax import lax
from jax.experimental import pallas as pl
from jax.experimental.pallas import tpu as pltpu
from jax.experimental.pallas import tpu_sc as plsc

M = 64
N = 64
B = 128
FLAT = M * N
NW = 32
SPW = B // NW
MP = 65
PFLAT = M * MP


def _mm_body(x_ref, w_ref, b_ref, o_ref):
    o_ref[...] = jnp.dot(
        x_ref[...], w_ref[...],
        preferred_element_type=jnp.float32,
    ) + b_ref[...]


def _matmul(x, W, b2d):
    NB = 8
    BN = FLAT // NB
    return pl.pallas_call(
        _mm_body,
        grid=(NB,),
        in_specs=[
            pl.BlockSpec((B, FLAT), lambda n: (0, 0)),
            pl.BlockSpec((FLAT, BN), lambda n: (0, n)),
            pl.BlockSpec((1, BN), lambda n: (0, n)),
        ],
        out_specs=pl.BlockSpec((B, BN), lambda n: (0, n)),
        out_shape=jax.ShapeDtypeStruct((B, FLAT), jnp.float32),
    )(x, W, b2d)


def _ce(ak, av, bk, bv):
    m = ak <= bk
    return (jnp.where(m, ak, bk), jnp.where(m, av, bv),
            jnp.where(m, bk, ak), jnp.where(m, bv, av))


def _mid32(s):
    lok, lov, hik, hiv = _ce(s[0][0], s[0][1], s[1][0], s[1][1])
    a0 = plsc.sort_key_val(lok, lov)
    a1 = plsc.sort_key_val(hik, hiv)
    lok, lov, hik, hiv = _ce(s[2][0], s[2][1], s[3][0], s[3][1])
    bd0 = plsc.sort_key_val(hik, hiv, descending=True)
    bd1 = plsc.sort_key_val(lok, lov, descending=True)
    return (a0, a1, bd0, bd1)


def _last32(mid):
    a0, a1, bd0, bd1 = mid
    l0k, l0v, h0k, h0v = _ce(a0[0], a0[1], bd0[0], bd0[1])
    l1k, l1v, h1k, h1v = _ce(a1[0], a1[1], bd1[0], bd1[1])
    llk, llv, lhk, lhv = _ce(l0k, l0v, l1k, l1v)
    hlk, hlv, hhk, hhv = _ce(h0k, h0v, h1k, h1v)
    outs = [plsc.sort_key_val(llk, llv), plsc.sort_key_val(lhk, lhv),
            plsc.sort_key_val(hlk, hlv), plsc.sort_key_val(hhk, hhv)]
    return [o[1] for o in outs]


def _finish64(s):
    return _last32(_mid32(s))


_SC_SCRATCH = [
    pltpu.VMEM((FLAT,), jnp.float32),
    pltpu.VMEM((FLAT,), jnp.float32),
    pltpu.VMEM((FLAT,), jnp.float32),
    pltpu.VMEM((FLAT,), jnp.float32),
    pltpu.VMEM((PFLAT,), jnp.float32),
    pltpu.VMEM((PFLAT,), jnp.int32),
    pltpu.VMEM((M, MP), jnp.float32),
    pltpu.VMEM((M, MP), jnp.float32),
    pltpu.SemaphoreType.DMA,
    pltpu.SemaphoreType.DMA,
    pltpu.SemaphoreType.DMA,
    pltpu.SemaphoreType.DMA,
]


def _sc_body(o_hbm, in_hbm, out_hbm,
             o_v0, o_v1, in_v0, in_v1, o_p, sx_p, res_p0, res_p1,
             sem_i0, sem_i1, sem_o0, sem_o1):
    wid = lax.axis_index("s") * 2 + lax.axis_index("c")
    iota = lax.iota(jnp.int32, 16)
    iotas = [iota + 16 * c for c in range(4)]
    o_vs = [o_v0, o_v1]
    in_vs = [in_v0, in_v1]
    res_ps = [res_p0, res_p1]
    sem_is = [sem_i0, sem_i1]
    sem_os = [sem_o0, sem_o1]
    s0 = wid * SPW

    def start_in(i, b):
        pltpu.async_copy(o_hbm.at[pl.ds((s0 + i) * FLAT, FLAT)],
                         o_vs[b], sem_is[b])
        pltpu.async_copy(in_hbm.at[pl.ds((s0 + i) * FLAT, FLAT)],
                         in_vs[b], sem_is[b])

    def wait_in(b):
        pltpu.make_async_copy(o_hbm.at[pl.ds(0, FLAT)], o_vs[b],
                              sem_is[b]).wait()
        pltpu.make_async_copy(in_hbm.at[pl.ds(0, FLAT)], in_vs[b],
                              sem_is[b]).wait()

    start_in(0, 0)

    for i in range(SPW):
        b = i % 2
        if i + 1 < SPW:
            start_in(i + 1, 1 - b)
        wait_in(b)
        o_v = o_vs[b]
        in_v = in_vs[b]
        res_p = res_ps[b]
        if i >= 2:
            pltpu.make_async_copy(
                res_p.at[:, pl.ds(0, M)],
                out_hbm.at[pl.ds(0, M)], sem_os[b]).wait()

        def init_row(r):
            ks = [o_v[pl.ds(r * 64 + 16 * c, 16)] for c in range(4)]
            for c in range(4):
                o_p[pl.ds(r * 65 + 16 * c, 16)] = ks[c]
            return tuple(plsc.sort_key_val(ks[c], iotas[c], descending=(c % 2 == 1))
                         for c in range(4))

        def finish_row(r, s):
            perm = _finish64(list(s))
            for c in range(4):
                sx_p[pl.ds(r * 65 + 16 * c, 16)] = perm[c]

        def row_body(r, carry):
            nxt = init_row(r + 1)
            finish_row(r, carry)
            return nxt
        rlast = lax.fori_loop(0, 63, row_body, init_row(0))
        finish_row(63, rlast)

        iota65 = [(iota + 16 * c) * 65 for c in range(4)]

        def init_sorts(j):
            ks = [plsc.load_gather(o_p, [iota65[c] + j]) for c in range(4)]
            return tuple(plsc.sort_key_val(ks[c], iotas[c], descending=(c % 2 == 1))
                         for c in range(4))

        def tail_col(j, mid):
            perm = _last32(mid)
            jv = iota * 0 + j
            for c in range(4):
                cc = plsc.load_gather(sx_p, [perm[c] * 65 + j])
                val = plsc.load_gather(in_v, [perm[c] * 64 + cc])
                plsc.store_scatter(res_p, [iotas[c], jv], val)

        def col_body(j, carry):
            s_next, mid_cur = carry
            s_new = init_sorts(j + 2)
            mid_next = _mid32(s_next)
            tail_col(j, mid_cur)
            return (s_new, mid_next)
        c0 = (init_sorts(1), _mid32(init_sorts(0)))
        s_last, mid_62 = lax.fori_loop(0, 62, col_body, c0)
        tail_col(62, mid_62)
        tail_col(63, _mid32(s_last))

        pltpu.async_copy(res_p.at[:, pl.ds(0, M)],
                         out_hbm.at[pl.ds((s0 + i) * M, M)], sem_os[b])

    for i in (SPW - 2, SPW - 1):
        b = i % 2
        pltpu.make_async_copy(res_ps[b].at[:, pl.ds(0, M)],
                              out_hbm.at[pl.ds(0, M)], sem_os[b]).wait()


_sc_permute = functools.partial(
    pl.kernel,
    out_type=jax.ShapeDtypeStruct((B * M, N), jnp.float32),
    mesh=plsc.VectorSubcoreMesh(
        core_axis_name="c", subcore_axis_name="s", num_cores=2, num_subcores=16),
    compiler_params=pltpu.CompilerParams(
        needs_layout_passes=False, use_tc_tiling_on_sc=False),
    scratch_types=_SC_SCRATCH,
)(_sc_body)


def kernel(input, W, b):
    x = jnp.reshape(input.astype(jnp.float32), (B, FLAT))
    o = _matmul(x, W, jnp.reshape(b, (1, FLAT)))
    res = _sc_permute(jnp.reshape(o, (B * FLAT,)), jnp.reshape(x, (B * FLAT,)))
    return jnp.reshape(res, (B, M, N))
